# baseline (speedup 1.0000x reference)
import functools

import jax
import jax.numpy as jnp
from jax import lax
from jax.experimental import pallas as pl


_NEG = -1e30


def _lrelu(v):
    return jnp.where(v >= 0, v, 0.2 * v)


def _mm_nt(a, b):
    return lax.dot_general(a, b, (((1,), (1,)), ((), ())),
                           preferred_element_type=jnp.float32)


def _flat_masks(s, R, S):
    idx = lax.broadcasted_iota(jnp.int32, (1, R * S), 1)
    p = s + idx // S
    q = idx % S
    ut = q > p
    m0 = ut & (p >= 1) & (p <= S - 2) & (q <= S - 2)
    m1 = ut & (p >= 1)
    m2 = (p >= 1) & (q > p + 1)
    m3 = (p >= 1) & (p <= S - 2) & (q > p + 1)
    return (m0, m1, m2, m3)


def _layer_kernel(x_ref, Wt_ref, ast_ref, adt_ref, b_ref, o_ref, *,
                  S, H, Dh, T, in_t, out_t):
    r = pl.program_id(1)
    nr = pl.num_programs(1)
    R = T + 2
    s = jnp.clip(r * T - 1, 0, S - R)

    if in_t:
        xT = x_ref[0, :, pl.ds(s * S, R * S)]
        hT = jnp.dot(Wt_ref[...], xT,
                     preferred_element_type=jnp.float32)
        asrcT = jnp.dot(ast_ref[...], xT,
                        preferred_element_type=jnp.float32)
        adstT = jnp.dot(adt_ref[...], xT,
                        preferred_element_type=jnp.float32)
    else:
        x2d = x_ref[0, pl.ds(s, R)].reshape(R * S, x_ref.shape[-1])
        hT = _mm_nt(Wt_ref[...], x2d)
        asrcT = _mm_nt(ast_ref[...], x2d)
        adstT = _mm_nt(adt_ref[...], x2d)

    masks = _flat_masks(s, R, S)
    rolls = (-(S + 1), S + 1, 1, -S)

    logits = [_lrelu(asrcT + adstT)]
    for ro, m in zip(rolls, masks):
        lg = _lrelu(jnp.roll(asrcT, ro, axis=1) + adstT)
        logits.append(jnp.where(m, lg, _NEG))
    mx = logits[0]
    for lg in logits[1:]:
        mx = jnp.maximum(mx, lg)
    es = [jnp.exp(lg - mx) for lg in logits]
    inv = 1.0 / (es[0] + es[1] + es[2] + es[3] + es[4])
    ws = [e * inv for e in es]

    accT = None
    for hd in range(H):
        hh = hT[hd * Dh:(hd + 1) * Dh]
        hp1 = jnp.roll(hh, 1, axis=1)
        hm1 = jnp.roll(hh, -1, axis=1)
        m = ws[0][hd:hd + 1] * hh
        m = m + ws[1][hd:hd + 1] * jnp.roll(hm1, -S, axis=1)
        m = m + ws[2][hd:hd + 1] * jnp.roll(hp1, S, axis=1)
        m = m + ws[3][hd:hd + 1] * hp1
        m = m + ws[4][hd:hd + 1] * jnp.roll(hh, -S, axis=1)
        accT = m if accT is None else accT + m

    yT = accT * (1.0 / H) + b_ref[...]

    def store(d):
        if out_t:
            o_ref[0] = yT[:, d * S:(d + T) * S]
        else:
            yt = yT[:, d * S:(d + T) * S]
            o_ref[0] = jnp.transpose(yt).reshape(T, S, Dh)

    @pl.when(r == 0)
    def _():
        store(0)

    @pl.when((r > 0) & (r < nr - 1))
    def _():
        store(1)

    @pl.when((r == nr - 1) & (nr > 1))
    def _():
        store(2)


def _gat_layer_call(x, Wt, ast, adt, bias, T, S, in_t, out_t):
    B = x.shape[0]
    H, _ = ast.shape
    Dh = Wt.shape[0] // H

    nix = len(x.shape) - 1
    in_x = pl.BlockSpec((1,) + x.shape[1:], lambda b, r: (b,) + (0,) * nix)
    if out_t:
        out_spec = pl.BlockSpec((1, Dh, T * S), lambda b, r: (b, 0, r))
        out_shape = jax.ShapeDtypeStruct((B, Dh, S * S), jnp.float32)
    else:
        out_spec = pl.BlockSpec((1, T, S, Dh), lambda b, r: (b, r, 0, 0))
        out_shape = jax.ShapeDtypeStruct((B, S, S, Dh), jnp.float32)

    return pl.pallas_call(
        functools.partial(_layer_kernel, S=S, H=H, Dh=Dh, T=T,
                          in_t=in_t, out_t=out_t),
        grid=(B, S // T),
        in_specs=[
            in_x,
            pl.BlockSpec(Wt.shape, lambda b, r: (0, 0)),
            pl.BlockSpec(ast.shape, lambda b, r: (0, 0)),
            pl.BlockSpec(adt.shape, lambda b, r: (0, 0)),
            pl.BlockSpec((Dh, 1), lambda b, r: (0, 0)),
        ],
        out_specs=out_spec,
        out_shape=out_shape,
    )(x, Wt, ast, adt, bias.reshape(-1, 1))


def _fold_attention(W, a_src, a_dst):
    H, Dh = a_src.shape
    Din = W.shape[0]
    Wh = W.reshape(Din, H, Dh)
    ast = jnp.einsum('dhc,hc->hd', Wh, a_src)
    adt = jnp.einsum('dhc,hc->hd', Wh, a_dst)
    return jnp.transpose(W), ast, adt


def kernel(node_embeddings, W1, a_src1, a_dst1, b1, W2, a_src2, a_dst2, b2):
    S = node_embeddings.shape[1]
    T = 32 if S % 32 == 0 and S >= 64 else (16 if S % 16 == 0 and S >= 32 else max(1, S // 4))
    Wt1, ast1, adt1 = _fold_attention(W1, a_src1, a_dst1)
    Wt2, ast2, adt2 = _fold_attention(W2, a_src2, a_dst2)
    y1 = _gat_layer_call(node_embeddings, Wt1, ast1, adt1, b1, T, S,
                         in_t=False, out_t=True)
    y2 = _gat_layer_call(y1, Wt2, ast2, adt2, b2, T, S,
                         in_t=True, out_t=False)
    return y2


# speedup vs baseline: 1.4215x; 1.4215x over previous
import functools

import jax
import jax.numpy as jnp
from jax import lax
from jax.experimental import pallas as pl


_NEG = -1e30


def _lrelu(v):
    return jnp.where(v >= 0, v, 0.2 * v)


def _mm_nt(a, b, out_dtype=jnp.float32):
    return lax.dot_general(a, b, (((1,), (1,)), ((), ())),
                           preferred_element_type=out_dtype)


def _flat_masks(s, R, S):
    idx = lax.broadcasted_iota(jnp.int32, (1, R * S), 1)
    p = s + idx // S
    q = idx % S
    ut = q > p
    m0 = ut & (p >= 1) & (p <= S - 2) & (q <= S - 2)
    m1 = ut & (p >= 1)
    m2 = (p >= 1) & (q > p + 1)
    m3 = (p >= 1) & (p <= S - 2) & (q > p + 1)
    return (m0, m1, m2, m3)


def _layer_kernel(x_ref, Wt_ref, ast_ref, adt_ref, b_ref, o_ref, *,
                  S, H, Dh, T, in_t, out_t):
    r = pl.program_id(1)
    nr = pl.num_programs(1)
    R = T + 2
    s = jnp.clip(r * T - 1, 0, S - R)

    if in_t:
        xT = x_ref[0, :, pl.ds(s * S, R * S)]
        hT = jnp.dot(Wt_ref[...], xT,
                     preferred_element_type=jnp.float32
                     ).astype(jnp.bfloat16)
        asrcT = jnp.dot(ast_ref[...], xT,
                        preferred_element_type=jnp.float32)
        adstT = jnp.dot(adt_ref[...], xT,
                        preferred_element_type=jnp.float32)
    else:
        x2d = x_ref[0, pl.ds(s, R)].reshape(R * S, x_ref.shape[-1])
        hT = _mm_nt(Wt_ref[...], x2d).astype(jnp.bfloat16)
        asrcT = _mm_nt(ast_ref[...], x2d)
        adstT = _mm_nt(adt_ref[...], x2d)

    masks = _flat_masks(s, R, S)
    rolls = (-(S + 1), S + 1, 1, -S)

    logits = [_lrelu(asrcT + adstT)]
    for ro, m in zip(rolls, masks):
        lg = _lrelu(jnp.roll(asrcT, ro, axis=1) + adstT)
        logits.append(jnp.where(m, lg, _NEG))
    mx = logits[0]
    for lg in logits[1:]:
        mx = jnp.maximum(mx, lg)
    es = [jnp.exp(lg - mx) for lg in logits]
    inv = 1.0 / (es[0] + es[1] + es[2] + es[3] + es[4])
    ws = [(e * inv).astype(jnp.bfloat16) for e in es]

    accT = None
    for hd in range(H):
        hh = hT[hd * Dh:(hd + 1) * Dh]
        hp1 = jnp.roll(hh, 1, axis=1)
        hm1 = jnp.roll(hh, -1, axis=1)
        m = ws[0][hd:hd + 1] * hh
        m = m + ws[1][hd:hd + 1] * jnp.roll(hm1, -S, axis=1)
        m = m + ws[2][hd:hd + 1] * jnp.roll(hp1, S, axis=1)
        m = m + ws[3][hd:hd + 1] * hp1
        m = m + ws[4][hd:hd + 1] * jnp.roll(hh, -S, axis=1)
        accT = m if accT is None else accT + m

    if out_t:
        yT = accT * jnp.bfloat16(1.0 / H) + b_ref[...]
    else:
        yT = accT.astype(jnp.float32) * (1.0 / H) + b_ref[...]

    def store(d):
        if out_t:
            o_ref[0] = yT[:, d * S:(d + T) * S]
        else:
            yt = yT[:, d * S:(d + T) * S]
            o_ref[0] = jnp.transpose(yt).reshape(T, S, Dh)

    @pl.when(r == 0)
    def _():
        store(0)

    @pl.when((r > 0) & (r < nr - 1))
    def _():
        store(1)

    @pl.when((r == nr - 1) & (nr > 1))
    def _():
        store(2)


def _gat_layer_call(x, Wt, ast, adt, bias, T, S, in_t, out_t):
    B = x.shape[0]
    H, _ = ast.shape
    Dh = Wt.shape[0] // H

    nix = len(x.shape) - 1
    in_x = pl.BlockSpec((1,) + x.shape[1:], lambda b, r: (b,) + (0,) * nix)
    if out_t:
        out_spec = pl.BlockSpec((1, Dh, T * S), lambda b, r: (b, 0, r))
        out_shape = jax.ShapeDtypeStruct((B, Dh, S * S), jnp.bfloat16)
    else:
        out_spec = pl.BlockSpec((1, T, S, Dh), lambda b, r: (b, r, 0, 0))
        out_shape = jax.ShapeDtypeStruct((B, S, S, Dh), jnp.float32)

    return pl.pallas_call(
        functools.partial(_layer_kernel, S=S, H=H, Dh=Dh, T=T,
                          in_t=in_t, out_t=out_t),
        grid=(B, S // T),
        in_specs=[
            in_x,
            pl.BlockSpec(Wt.shape, lambda b, r: (0, 0)),
            pl.BlockSpec(ast.shape, lambda b, r: (0, 0)),
            pl.BlockSpec(adt.shape, lambda b, r: (0, 0)),
            pl.BlockSpec((Dh, 1), lambda b, r: (0, 0)),
        ],
        out_specs=out_spec,
        out_shape=out_shape,
    )(x, Wt, ast, adt,
      bias.reshape(-1, 1).astype(jnp.bfloat16 if out_t else jnp.float32))


def _fold_attention(W, a_src, a_dst):
    H, Dh = a_src.shape
    Din = W.shape[0]
    Wh = W.reshape(Din, H, Dh)
    ast = jnp.einsum('dhc,hc->hd', Wh, a_src)
    adt = jnp.einsum('dhc,hc->hd', Wh, a_dst)
    return jnp.transpose(W), ast, adt


def kernel(node_embeddings, W1, a_src1, a_dst1, b1, W2, a_src2, a_dst2, b2):
    S = node_embeddings.shape[1]
    T = 32 if S % 32 == 0 and S >= 64 else (16 if S % 16 == 0 and S >= 32 else max(1, S // 4))
    Wt1, ast1, adt1 = _fold_attention(W1, a_src1, a_dst1)
    Wt2, ast2, adt2 = _fold_attention(W2, a_src2, a_dst2)
    xb = node_embeddings.astype(jnp.bfloat16)
    Wt1, ast1, adt1, Wt2, ast2, adt2 = (
        a.astype(jnp.bfloat16) for a in (Wt1, ast1, adt1, Wt2, ast2, adt2))
    y1 = _gat_layer_call(xb, Wt1, ast1, adt1, b1, T, S,
                         in_t=False, out_t=True)
    y2 = _gat_layer_call(y1, Wt2, ast2, adt2, b2, T, S,
                         in_t=True, out_t=False)
    return y2


# speedup vs baseline: 1.4819x; 1.0425x over previous
import functools

import jax
import jax.numpy as jnp
from jax import lax
from jax.experimental import pallas as pl


_NEG = -1e30


def _lrelu(v):
    return jnp.where(v >= 0, v, 0.2 * v)


def _mm_nt(a, b, out_dtype=jnp.float32):
    return lax.dot_general(a, b, (((1,), (1,)), ((), ())),
                           preferred_element_type=out_dtype)


def _flat_masks(s, R, S):
    idx = lax.broadcasted_iota(jnp.int32, (1, R * S), 1)
    p = s + idx // S
    q = idx % S
    ut = q > p
    m0 = ut & (p >= 1) & (p <= S - 2) & (q <= S - 2)
    m1 = ut & (p >= 1)
    m2 = (p >= 1) & (q > p + 1)
    m3 = (p >= 1) & (p <= S - 2) & (q > p + 1)
    return (m0, m1, m2, m3)


def _layer_kernel(x_ref, Wt_ref, ast_ref, adt_ref, b_ref, o_ref, *,
                  S, H, Dh, T, in_t, out_t):
    r = pl.program_id(1)
    nr = pl.num_programs(1)
    R = T + 2
    s = jnp.clip(r * T - 1, 0, S - R)

    if in_t:
        xT = x_ref[0, :, pl.ds(s * S, R * S)]
        hT = jnp.dot(Wt_ref[...], xT,
                     preferred_element_type=jnp.float32
                     ).astype(jnp.bfloat16)
        asrcT = jnp.dot(ast_ref[...], xT,
                        preferred_element_type=jnp.float32)
        adstT = jnp.dot(adt_ref[...], xT,
                        preferred_element_type=jnp.float32)
    else:
        x2d = x_ref[0, pl.ds(s, R)].reshape(R * S, x_ref.shape[-1])
        hT = _mm_nt(Wt_ref[...], x2d).astype(jnp.bfloat16)
        asrcT = _mm_nt(ast_ref[...], x2d)
        adstT = _mm_nt(adt_ref[...], x2d)

    masks = _flat_masks(s, R, S)
    rolls = (-(S + 1), S + 1, 1, -S)

    logits = [_lrelu(asrcT + adstT)]
    for ro, m in zip(rolls, masks):
        lg = _lrelu(jnp.roll(asrcT, ro, axis=1) + adstT)
        logits.append(jnp.where(m, lg, _NEG))
    mx = logits[0]
    for lg in logits[1:]:
        mx = jnp.maximum(mx, lg)
    es = [jnp.exp(lg - mx) for lg in logits]
    inv = 1.0 / (es[0] + es[1] + es[2] + es[3] + es[4])
    ws = [(e * inv).astype(jnp.bfloat16) for e in es]

    accT = None
    for hd in range(H):
        hh = hT[hd * Dh:(hd + 1) * Dh]
        hp1 = jnp.roll(hh, 1, axis=1)
        hm1 = jnp.roll(hh, -1, axis=1)
        m = ws[0][hd:hd + 1] * hh
        m = m + ws[1][hd:hd + 1] * jnp.roll(hm1, -S, axis=1)
        m = m + ws[2][hd:hd + 1] * jnp.roll(hp1, S, axis=1)
        m = m + ws[3][hd:hd + 1] * hp1
        m = m + ws[4][hd:hd + 1] * jnp.roll(hh, -S, axis=1)
        accT = m if accT is None else accT + m

    if out_t:
        yT = accT * jnp.bfloat16(1.0 / H) + b_ref[...]
    else:
        yT = accT.astype(jnp.float32) * (1.0 / H) + b_ref[...]

    def store(d):
        if out_t:
            o_ref[0] = yT[:, d * S:(d + T) * S]
        else:
            yt = yT[:, d * S:(d + T) * S]
            o_ref[0] = jnp.transpose(yt).reshape(T, S, Dh)

    @pl.when(r == 0)
    def _():
        store(0)

    @pl.when((r > 0) & (r < nr - 1))
    def _():
        store(1)

    @pl.when((r == nr - 1) & (nr > 1))
    def _():
        store(2)


def _gat_layer_call(x, Wt, ast, adt, bias, T, S, in_t, out_t):
    B = x.shape[0]
    H, _ = ast.shape
    Dh = Wt.shape[0] // H

    nix = len(x.shape) - 1
    in_x = pl.BlockSpec((1,) + x.shape[1:], lambda b, r: (b,) + (0,) * nix)
    if out_t:
        out_spec = pl.BlockSpec((1, Dh, T * S), lambda b, r: (b, 0, r))
        out_shape = jax.ShapeDtypeStruct((B, Dh, S * S), jnp.bfloat16)
    else:
        out_spec = pl.BlockSpec((1, T, S, Dh), lambda b, r: (b, r, 0, 0))
        out_shape = jax.ShapeDtypeStruct((B, S, S, Dh), jnp.float32)

    return pl.pallas_call(
        functools.partial(_layer_kernel, S=S, H=H, Dh=Dh, T=T,
                          in_t=in_t, out_t=out_t),
        grid=(B, S // T),
        in_specs=[
            in_x,
            pl.BlockSpec(Wt.shape, lambda b, r: (0, 0)),
            pl.BlockSpec(ast.shape, lambda b, r: (0, 0)),
            pl.BlockSpec(adt.shape, lambda b, r: (0, 0)),
            pl.BlockSpec((Dh, 1), lambda b, r: (0, 0)),
        ],
        out_specs=out_spec,
        out_shape=out_shape,
    )(x, Wt, ast, adt,
      bias.reshape(-1, 1).astype(jnp.bfloat16 if out_t else jnp.float32))


def _fold_attention(W, a_src, a_dst):
    H, Dh = a_src.shape
    Din = W.shape[0]
    Wh = W.reshape(Din, H, Dh)
    ast = jnp.einsum('dhc,hc->hd', Wh, a_src)
    adt = jnp.einsum('dhc,hc->hd', Wh, a_dst)
    return jnp.transpose(W), ast, adt


def kernel(node_embeddings, W1, a_src1, a_dst1, b1, W2, a_src2, a_dst2, b2):
    S = node_embeddings.shape[1]
    T = next((t for t in (64, 32, 16) if S % t == 0 and S >= t + 2 and S // t >= 2),
             max(1, S // 4))
    Wt1, ast1, adt1 = _fold_attention(W1, a_src1, a_dst1)
    Wt2, ast2, adt2 = _fold_attention(W2, a_src2, a_dst2)
    xb = node_embeddings.astype(jnp.bfloat16)
    Wt1, ast1, adt1, Wt2, ast2, adt2 = (
        a.astype(jnp.bfloat16) for a in (Wt1, ast1, adt1, Wt2, ast2, adt2))
    y1 = _gat_layer_call(xb, Wt1, ast1, adt1, b1, T, S,
                         in_t=False, out_t=True)
    y2 = _gat_layer_call(y1, Wt2, ast2, adt2, b2, T, S,
                         in_t=True, out_t=False)
    return y2
